# baseline (speedup 1.0000x reference)
import functools

import jax
import jax.numpy as jnp
from jax import lax
from jax.experimental import pallas as pl
from jax.experimental.pallas import tpu as pltpu
from jax.experimental.pallas import tpu_sc as plsc

S = 2048
DM = 768
FF = 2048
E = 8
K = 2
BM = 384
NB = (K * S + E * (BM - 1) + BM - 1) // BM
PT = NB * BM
NBP = 32
NW = 32


def _router_body(x_ref, wr_ref, d_ref, g_ref, be_ref, bv_ref):
    xb = x_ref[...]
    logits = jnp.dot(xb, wr_ref[...])
    m1 = jnp.max(logits, axis=1, keepdims=True)
    i1 = jnp.argmax(logits, axis=1, keepdims=True)
    lanes = lax.broadcasted_iota(jnp.int32, logits.shape, 1)
    masked = jnp.where(lanes == i1, -jnp.inf, logits)
    m2 = jnp.max(masked, axis=1, keepdims=True)
    i2 = jnp.argmax(masked, axis=1, keepdims=True)
    g1 = 1.0 / (1.0 + jnp.exp(m2 - m1))
    g_ref[...] = jnp.concatenate([g1, 1.0 - g1], axis=1)

    e2d = jnp.concatenate([i1.reshape(16, 128), i2.reshape(16, 128)], axis=0)
    oh = (e2d[:, None, :] == lax.broadcasted_iota(jnp.int32, (1, E, 1), 1))
    oh = oh.astype(jnp.float32)
    oh2 = oh.reshape(32 * E, 128)

    ib = lax.broadcasted_iota(jnp.int32, (128, 128), 0)
    ic = lax.broadcasted_iota(jnp.int32, (128, 128), 1)
    triu128 = (ib < ic).astype(jnp.float32)
    r2 = jnp.dot(oh2, triu128)

    s_ae = jnp.dot(oh2, jnp.ones((128, 1), jnp.float32)).reshape(32, E)
    ia = lax.broadcasted_iota(jnp.int32, (32, 32), 0)
    ja = lax.broadcasted_iota(jnp.int32, (32, 32), 1)
    tril32 = (ja < ia).astype(jnp.float32)
    cs = jnp.dot(tril32, s_ae)

    counts = jnp.dot(jnp.ones((1, 32), jnp.float32), s_ae)
    pc = jnp.ceil(counts / BM) * BM
    ie = lax.broadcasted_iota(jnp.int32, (E, E), 0)
    je = lax.broadcasted_iota(jnp.int32, (E, E), 1)
    triu8 = (ie < je).astype(jnp.float32)
    poffs = jnp.dot(pc, triu8)
    pend = poffs + pc

    rank3 = r2.reshape(32, E, 128) + cs[:, :, None] + poffs.reshape(1, E, 1)
    d3 = jnp.sum(rank3 * oh, axis=1)
    d_ref[...] = d3.astype(jnp.int32)

    rstart = lax.broadcasted_iota(jnp.int32, (NBP, 1), 0).astype(jnp.float32) * BM
    bexp = jnp.sum((rstart >= pend).astype(jnp.float32), axis=1, keepdims=True)
    total = jnp.dot(pc, jnp.ones((E, 1), jnp.float32))
    bval = (rstart < total[0, 0]).astype(jnp.int32)
    bexp = jnp.minimum(bexp, float(E - 1)).astype(jnp.int32)
    be_ref[...] = jnp.where(bval == 1, bexp, 0)
    bv_ref[...] = bval


def _run_router(x2, W_router):
    return pl.pallas_call(
        _router_body,
        out_shape=[
            jax.ShapeDtypeStruct((32, 128), jnp.int32),
            jax.ShapeDtypeStruct((S, K), jnp.float32),
            jax.ShapeDtypeStruct((NBP, 1), jnp.int32),
            jax.ShapeDtypeStruct((NBP, 1), jnp.int32),
        ],
    )(x2, W_router)


def _sc_dispatch_body(x_hbm, d_hbm, xs_hbm, idx0_v, idx1_v, rows_v, sem):
    wid = lax.axis_index("s") * 2 + lax.axis_index("c")
    ct = S // NW
    base = wid * ct
    pltpu.sync_copy(x_hbm.at[pl.ds(base, ct)], rows_v)
    pltpu.sync_copy(d_hbm.at[pl.ds(base, ct)], idx0_v)
    pltpu.sync_copy(d_hbm.at[pl.ds(S + base, ct)], idx1_v)
    pltpu.async_copy(rows_v, xs_hbm.at[idx0_v], sem).wait()
    pltpu.async_copy(rows_v, xs_hbm.at[idx1_v], sem).wait()


def _run_dispatch(x2, d_flat):
    ct = S // NW
    mesh = plsc.VectorSubcoreMesh(core_axis_name="c", subcore_axis_name="s")
    f = functools.partial(
        pl.kernel,
        out_type=jax.ShapeDtypeStruct((PT, DM), jnp.float32),
        mesh=mesh,
        scratch_types=[
            pltpu.VMEM((ct,), jnp.int32),
            pltpu.VMEM((ct,), jnp.int32),
            pltpu.VMEM((ct, DM), jnp.float32),
            pltpu.SemaphoreType.DMA,
        ],
    )(_sc_dispatch_body)
    return f(x2, d_flat)


def _ffn_body(be_ref, bv_ref, xs_ref, w1_ref, b1_ref, w2_ref, b2_ref, y_ref):
    r = pl.program_id(0)

    @pl.when(bv_ref[r] == 1)
    def _():
        xb = xs_ref[...].astype(jnp.bfloat16)
        w1 = w1_ref[0].astype(jnp.bfloat16)
        h = jnp.dot(xb, w1, preferred_element_type=jnp.float32)
        h = jnp.maximum(h + b1_ref[0], 0.0).astype(jnp.bfloat16)
        w2 = w2_ref[0].astype(jnp.bfloat16)
        y = jnp.dot(h, w2, preferred_element_type=jnp.float32)
        y_ref[...] = y + b2_ref[0]


def _run_ffn(xs, W1, b1r, W2, b2r, bexp, bval):
    grid_spec = pltpu.PrefetchScalarGridSpec(
        num_scalar_prefetch=2,
        grid=(NB,),
        in_specs=[
            pl.BlockSpec((BM, DM), lambda r, be, bv: (r, 0)),
            pl.BlockSpec((1, DM, FF), lambda r, be, bv: (be[r], 0, 0)),
            pl.BlockSpec((1, 1, FF), lambda r, be, bv: (be[r], 0, 0)),
            pl.BlockSpec((1, FF, DM), lambda r, be, bv: (be[r], 0, 0)),
            pl.BlockSpec((1, 1, DM), lambda r, be, bv: (be[r], 0, 0)),
        ],
        out_specs=pl.BlockSpec((BM, DM), lambda r, be, bv: (r, 0)),
    )
    return pl.pallas_call(
        _ffn_body,
        grid_spec=grid_spec,
        out_shape=jax.ShapeDtypeStruct((PT, DM), jnp.float32),
    )(bexp, bval, xs, W1, b1r, W2, b2r)


def _sc_gather_body(y_hbm, d_hbm, ys_hbm, idx_v, rows_v, sem):
    wid = lax.axis_index("s") * 2 + lax.axis_index("c")
    cp = (K * S) // NW
    base = wid * cp
    pltpu.sync_copy(d_hbm.at[pl.ds(base, cp)], idx_v)
    pltpu.async_copy(y_hbm.at[idx_v], rows_v, sem).wait()
    pltpu.sync_copy(rows_v, ys_hbm.at[pl.ds(base, cp)])


def _run_gather(y, d_flat):
    cp = (K * S) // NW
    mesh = plsc.VectorSubcoreMesh(core_axis_name="c", subcore_axis_name="s")
    f = functools.partial(
        pl.kernel,
        out_type=jax.ShapeDtypeStruct((K * S, DM), jnp.float32),
        mesh=mesh,
        scratch_types=[
            pltpu.VMEM((cp,), jnp.int32),
            pltpu.VMEM((cp, DM), jnp.float32),
            pltpu.SemaphoreType.DMA,
        ],
    )(_sc_gather_body)
    return f(y, d_flat)


def _combine_body(g_ref, ys0_ref, ys1_ref, out_ref):
    g = g_ref[...]
    out_ref[...] = ys0_ref[...] * g[:, 0:1] + ys1_ref[...] * g[:, 1:2]


def _run_combine(gates, ys):
    BT = 512
    nb0 = S // BT
    return pl.pallas_call(
        _combine_body,
        grid=(nb0,),
        in_specs=[
            pl.BlockSpec((BT, K), lambda t: (t, 0)),
            pl.BlockSpec((BT, DM), lambda t: (t, 0)),
            pl.BlockSpec((BT, DM), lambda t: (t + nb0, 0)),
        ],
        out_specs=pl.BlockSpec((BT, DM), lambda t: (t, 0)),
        out_shape=jax.ShapeDtypeStruct((S, DM), jnp.float32),
    )(gates, ys, ys)


def kernel(x, W_router, W1, b1, W2, b2):
    B = x.shape[0]
    x2 = x.reshape(S, DM)
    b1r = b1.reshape(E, 1, FF)
    b2r = b2.reshape(E, 1, DM)

    d, gates, bexp, bval = _run_router(x2, W_router)
    d_flat = d.reshape(K * S)
    bexp_f = bexp.reshape(NBP)[:NB]
    bval_f = bval.reshape(NBP)[:NB]

    xs = _run_dispatch(x2, d_flat)
    y = _run_ffn(xs, W1, b1r, W2, b2r, bexp_f, bval_f)
    ys = _run_gather(y, d_flat)
    out = _run_combine(gates, ys)
    return out.reshape(B, S, DM)


# speedup vs baseline: 1.1371x; 1.1371x over previous
import functools

import jax
import jax.numpy as jnp
from jax import lax
from jax.experimental import pallas as pl
from jax.experimental.pallas import tpu as pltpu
from jax.experimental.pallas import tpu_sc as plsc

S = 2048
DM = 768
FF = 2048
E = 8
K = 2
BM = 512
NB = (K * S + E * (BM - 1) + BM - 1) // BM
PT = NB * BM
NBP = 32
NW = 32


def _router_body(x_ref, wr_ref, d_ref, g_ref, be_ref, bv_ref):
    xb = x_ref[...]
    logits = jnp.dot(xb, wr_ref[...])
    m1 = jnp.max(logits, axis=1, keepdims=True)
    i1 = jnp.argmax(logits, axis=1, keepdims=True)
    lanes = lax.broadcasted_iota(jnp.int32, logits.shape, 1)
    masked = jnp.where(lanes == i1, -jnp.inf, logits)
    m2 = jnp.max(masked, axis=1, keepdims=True)
    i2 = jnp.argmax(masked, axis=1, keepdims=True)
    g1 = 1.0 / (1.0 + jnp.exp(m2 - m1))
    g_ref[...] = jnp.concatenate([g1, 1.0 - g1], axis=1)

    e2d = jnp.concatenate([i1.reshape(16, 128), i2.reshape(16, 128)], axis=0)
    oh = (e2d[:, None, :] == lax.broadcasted_iota(jnp.int32, (1, E, 1), 1))
    oh = oh.astype(jnp.float32)
    oh2 = oh.reshape(32 * E, 128)

    ib = lax.broadcasted_iota(jnp.int32, (128, 128), 0)
    ic = lax.broadcasted_iota(jnp.int32, (128, 128), 1)
    triu128 = (ib < ic).astype(jnp.float32)
    r2 = jnp.dot(oh2, triu128)

    s_ae = jnp.dot(oh2, jnp.ones((128, 1), jnp.float32)).reshape(32, E)
    ia = lax.broadcasted_iota(jnp.int32, (32, 32), 0)
    ja = lax.broadcasted_iota(jnp.int32, (32, 32), 1)
    tril32 = (ja < ia).astype(jnp.float32)
    cs = jnp.dot(tril32, s_ae)

    counts = jnp.dot(jnp.ones((1, 32), jnp.float32), s_ae)
    pc = jnp.ceil(counts / BM) * BM
    ie = lax.broadcasted_iota(jnp.int32, (E, E), 0)
    je = lax.broadcasted_iota(jnp.int32, (E, E), 1)
    triu8 = (ie < je).astype(jnp.float32)
    poffs = jnp.dot(pc, triu8)
    pend = poffs + pc

    rank3 = r2.reshape(32, E, 128) + cs[:, :, None] + poffs.reshape(1, E, 1)
    d3 = jnp.sum(rank3 * oh, axis=1)
    d_ref[...] = d3.astype(jnp.int32)

    rstart = lax.broadcasted_iota(jnp.int32, (NBP, 1), 0).astype(jnp.float32) * BM
    bexp = jnp.sum((rstart >= pend).astype(jnp.float32), axis=1, keepdims=True)
    total = jnp.dot(pc, jnp.ones((E, 1), jnp.float32))
    bval = (rstart < total[0, 0]).astype(jnp.int32)
    bexp = jnp.minimum(bexp, float(E - 1)).astype(jnp.int32)
    be_ref[...] = bexp
    bv_ref[...] = bval


def _run_router(x2, W_router):
    return pl.pallas_call(
        _router_body,
        out_shape=[
            jax.ShapeDtypeStruct((32, 128), jnp.int32),
            jax.ShapeDtypeStruct((S, K), jnp.float32),
            jax.ShapeDtypeStruct((NBP, 1), jnp.int32),
            jax.ShapeDtypeStruct((NBP, 1), jnp.int32),
        ],
    )(x2, W_router)


def _sc_dispatch_body(x_hbm, d_hbm, xs_hbm, idx0_v, idx1_v, rows_v, sem):
    wid = lax.axis_index("s") * 2 + lax.axis_index("c")
    ct = S // NW
    base = wid * ct
    pltpu.sync_copy(x_hbm.at[pl.ds(base, ct)], rows_v)
    pltpu.sync_copy(d_hbm.at[pl.ds(base, ct)], idx0_v)
    pltpu.sync_copy(d_hbm.at[pl.ds(S + base, ct)], idx1_v)
    pltpu.async_copy(rows_v, xs_hbm.at[idx0_v], sem).wait()
    pltpu.async_copy(rows_v, xs_hbm.at[idx1_v], sem).wait()


def _run_dispatch(x2, d_flat):
    ct = S // NW
    mesh = plsc.VectorSubcoreMesh(core_axis_name="c", subcore_axis_name="s")
    f = functools.partial(
        pl.kernel,
        out_type=jax.ShapeDtypeStruct((PT, DM), jnp.float32),
        mesh=mesh,
        scratch_types=[
            pltpu.VMEM((ct,), jnp.int32),
            pltpu.VMEM((ct,), jnp.int32),
            pltpu.VMEM((ct, DM), jnp.float32),
            pltpu.SemaphoreType.DMA,
        ],
    )(_sc_dispatch_body)
    return f(x2, d_flat)


def _ffn_body(be_ref, bv_ref, xs_ref, w1_ref, b1_ref, w2_ref, b2_ref, y_ref):
    r = pl.program_id(0)

    @pl.when(bv_ref[r] == 1)
    def _():
        xb = xs_ref[...].astype(jnp.bfloat16)
        w1 = w1_ref[0].astype(jnp.bfloat16)
        h = jnp.dot(xb, w1, preferred_element_type=jnp.float32)
        h = jnp.maximum(h + b1_ref[0], 0.0).astype(jnp.bfloat16)
        w2 = w2_ref[0].astype(jnp.bfloat16)
        y = jnp.dot(h, w2, preferred_element_type=jnp.float32)
        y_ref[...] = y + b2_ref[0]


def _run_ffn(xs, W1, b1r, W2, b2r, bexp, bval):
    grid_spec = pltpu.PrefetchScalarGridSpec(
        num_scalar_prefetch=2,
        grid=(NB,),
        in_specs=[
            pl.BlockSpec((BM, DM),
                         lambda r, be, bv: (jnp.where(bv[r] == 1, r, NB - 1), 0)),
            pl.BlockSpec((1, DM, FF), lambda r, be, bv: (be[r], 0, 0)),
            pl.BlockSpec((1, 1, FF), lambda r, be, bv: (be[r], 0, 0)),
            pl.BlockSpec((1, FF, DM), lambda r, be, bv: (be[r], 0, 0)),
            pl.BlockSpec((1, 1, DM), lambda r, be, bv: (be[r], 0, 0)),
        ],
        out_specs=pl.BlockSpec(
            (BM, DM), lambda r, be, bv: (jnp.where(bv[r] == 1, r, NB - 1), 0)),
    )
    return pl.pallas_call(
        _ffn_body,
        grid_spec=grid_spec,
        out_shape=jax.ShapeDtypeStruct((PT, DM), jnp.float32),
    )(bexp, bval, xs, W1, b1r, W2, b2r)


def _sc_gather_body(y_hbm, d_hbm, ys_hbm, idx_v, rows_v, sem):
    wid = lax.axis_index("s") * 2 + lax.axis_index("c")
    cp = (K * S) // NW
    base = wid * cp
    pltpu.sync_copy(d_hbm.at[pl.ds(base, cp)], idx_v)
    pltpu.async_copy(y_hbm.at[idx_v], rows_v, sem).wait()
    pltpu.sync_copy(rows_v, ys_hbm.at[pl.ds(base, cp)])


def _run_gather(y, d_flat):
    cp = (K * S) // NW
    mesh = plsc.VectorSubcoreMesh(core_axis_name="c", subcore_axis_name="s")
    f = functools.partial(
        pl.kernel,
        out_type=jax.ShapeDtypeStruct((K * S, DM), jnp.float32),
        mesh=mesh,
        scratch_types=[
            pltpu.VMEM((cp,), jnp.int32),
            pltpu.VMEM((cp, DM), jnp.float32),
            pltpu.SemaphoreType.DMA,
        ],
    )(_sc_gather_body)
    return f(y, d_flat)


def _combine_body(g_ref, ys0_ref, ys1_ref, out_ref):
    g = g_ref[...]
    out_ref[...] = ys0_ref[...] * g[:, 0:1] + ys1_ref[...] * g[:, 1:2]


def _run_combine(gates, ys):
    BT = 512
    nb0 = S // BT
    return pl.pallas_call(
        _combine_body,
        grid=(nb0,),
        in_specs=[
            pl.BlockSpec((BT, K), lambda t: (t, 0)),
            pl.BlockSpec((BT, DM), lambda t: (t, 0)),
            pl.BlockSpec((BT, DM), lambda t: (t + nb0, 0)),
        ],
        out_specs=pl.BlockSpec((BT, DM), lambda t: (t, 0)),
        out_shape=jax.ShapeDtypeStruct((S, DM), jnp.float32),
    )(gates, ys, ys)


def kernel(x, W_router, W1, b1, W2, b2):
    B = x.shape[0]
    x2 = x.reshape(S, DM)
    b1r = b1.reshape(E, 1, FF)
    b2r = b2.reshape(E, 1, DM)

    d, gates, bexp, bval = _run_router(x2, W_router)
    d_flat = d.reshape(K * S)
    bexp_f = bexp.reshape(NBP)[:NB]
    bval_f = bval.reshape(NBP)[:NB]

    xs = _run_dispatch(x2, d_flat)
    y = _run_ffn(xs, W1, b1r, W2, b2r, bexp_f, bval_f)
    ys = _run_gather(y, d_flat)
    out = _run_combine(gates, ys)
    return out.reshape(B, S, DM)
